# Initial kernel scaffold; baseline (speedup 1.0000x reference)
import jax
import jax.numpy as jnp
from jax.experimental import pallas as pl


def kernel(input_vector, table):
    raise NotImplementedError("write your pallas kernel here")


# speedup vs baseline: 1.7785x; 1.7785x over previous
import functools

import jax
import jax.numpy as jnp
from jax import lax
from jax.experimental import pallas as pl
from jax.experimental.pallas import tpu as pltpu
from jax.experimental.pallas import tpu_sc as plsc

NUM_EMB = 1_000_000
D = 64
B = 16384 * 50
NC, NS = 2, 16
NW = NC * NS
B_PER_W = B // NW
IW = 128
CH = 256
N_CHUNKS = B_PER_W // CH
IDX_ROWS = B_PER_W // IW


def _gather(table, idx):
    mesh = plsc.VectorSubcoreMesh(core_axis_name="c", subcore_axis_name="s")

    @functools.partial(
        pl.kernel,
        out_type=jax.ShapeDtypeStruct((B, D), jnp.float32),
        mesh=mesh,
        scratch_types=[
            pltpu.VMEM((IDX_ROWS, IW), jnp.int32),
            pltpu.VMEM((CH, D), jnp.float32),
            pltpu.SemaphoreType.DMA,
        ],
        compiler_params=pltpu.CompilerParams(use_tc_tiling_on_sc=False),
    )
    def k(idx_hbm, table_hbm, out_hbm, idx_v, rows_v, sem):
        wid = lax.axis_index("s") * NC + lax.axis_index("c")
        base = wid * B_PER_W
        pltpu.sync_copy(idx_hbm.at[wid], idx_v)

        @pl.loop(0, N_CHUNKS)
        def chunk(c):
            g0 = pltpu.async_copy(
                table_hbm.at[idx_v.at[2 * c]], rows_v.at[pl.ds(0, IW)], sem)
            g1 = pltpu.async_copy(
                table_hbm.at[idx_v.at[2 * c + 1]], rows_v.at[pl.ds(IW, IW)], sem)
            g0.wait()
            g1.wait()
            pltpu.sync_copy(rows_v, out_hbm.at[pl.ds(base + c * CH, CH)])

    return k(idx, table)


@jax.jit
def kernel(input_vector, table):
    idx = input_vector.reshape(NW, IDX_ROWS, IW)
    out = _gather(table, idx)
    return out.reshape(input_vector.shape + (D,))


# speedup vs baseline: 1.8738x; 1.0536x over previous
import functools

import jax
import jax.numpy as jnp
from jax import lax
from jax.experimental import pallas as pl
from jax.experimental.pallas import tpu as pltpu
from jax.experimental.pallas import tpu_sc as plsc

NUM_EMB = 1_000_000
D = 64
B = 16384 * 50
NC, NS = 2, 16
NW = NC * NS
B_PER_W = B // NW
IW = 128
CH = 256
N_CHUNKS = B_PER_W // CH
IDX_ROWS = B_PER_W // IW
NBUF = 4
LA = 2


def _gather(table, idx):
    mesh = plsc.VectorSubcoreMesh(core_axis_name="c", subcore_axis_name="s")

    @functools.partial(
        pl.kernel,
        out_type=jax.ShapeDtypeStruct((B, D), jnp.float32),
        mesh=mesh,
        scratch_types=[
            pltpu.VMEM((IDX_ROWS, IW), jnp.int32),
            pltpu.VMEM((NBUF, CH, D), jnp.float32),
            pltpu.SemaphoreType.DMA((NBUF,)),
            pltpu.SemaphoreType.DMA((NBUF,)),
        ],
        compiler_params=pltpu.CompilerParams(use_tc_tiling_on_sc=False),
    )
    def k(idx_hbm, table_hbm, out_hbm, idx_v, rows_v, sem_g, sem_w):
        wid = lax.axis_index("s") * NC + lax.axis_index("c")
        base = wid * B_PER_W
        pltpu.sync_copy(idx_hbm.at[wid], idx_v)

        def fire_gather(c, b):
            pltpu.async_copy(
                table_hbm.at[idx_v.at[2 * c]],
                rows_v.at[b, pl.ds(0, IW)], sem_g.at[b])
            pltpu.async_copy(
                table_hbm.at[idx_v.at[2 * c + 1]],
                rows_v.at[b, pl.ds(IW, IW)], sem_g.at[b])

        def wait_gather(b):
            pltpu.make_async_copy(
                out_hbm.at[pl.ds(0, CH)], rows_v.at[b], sem_g.at[b]).wait()

        def fire_wb(c, b):
            pltpu.async_copy(
                rows_v.at[b], out_hbm.at[pl.ds(base + c * CH, CH)],
                sem_w.at[b])

        def wait_wb(b):
            pltpu.make_async_copy(
                rows_v.at[b], out_hbm.at[pl.ds(0, CH)], sem_w.at[b]).wait()

        for c in range(LA):
            fire_gather(c, c % NBUF)

        for c in range(NBUF):
            if c - (NBUF - LA) >= 0:
                wait_wb((c + LA) % NBUF)
            fire_gather(c + LA, (c + LA) % NBUF)
            wait_gather(c % NBUF)
            fire_wb(c, c % NBUF)

        @pl.loop(NBUF, N_CHUNKS - NBUF, step=NBUF)
        def steady(c0):
            for b in range(NBUF):
                c = c0 + b
                wait_wb((b + LA) % NBUF)
                fire_gather(c + LA, (b + LA) % NBUF)
                wait_gather(b)
                fire_wb(c, b)

        for c in range(N_CHUNKS - NBUF, N_CHUNKS):
            b = c % NBUF
            if c + LA < N_CHUNKS:
                wait_wb((b + LA) % NBUF)
                fire_gather(c + LA, (b + LA) % NBUF)
            wait_gather(b)
            fire_wb(c, b)

        for b in range(NBUF):
            wait_wb(b)

    return k(idx, table)


@jax.jit
def kernel(input_vector, table):
    idx = input_vector.reshape(NW, IDX_ROWS, IW)
    out = _gather(table, idx)
    return out.reshape(input_vector.shape + (D,))
